# baseline (speedup 1.0000x reference)
import functools
import math

import jax
import jax.numpy as jnp
from jax import lax
from jax.experimental import pallas as pl
from jax.experimental.pallas import tpu as pltpu
from jax.experimental.pallas import tpu_sc as plsc

N, L = 1024, 50
B = N * L
D = 128
N_BANDS = 8

_NC, _NS = 2, 16
_NW = _NC * _NS
_BPW = B // _NW
_HALF = 800
_CH = 80
_NFIRE = _HALF // _CH
_NCH = _BPW // _CH


def _fourier_w(n_bands, offset):
    steps = n_bands + offset + 1
    w = 2.0 ** jnp.linspace(-float(n_bands), float(offset), steps)
    return (w * math.pi).astype(jnp.float32)


def _make_sc_gather():
    mesh = plsc.VectorSubcoreMesh(core_axis_name="c", subcore_axis_name="s")

    @functools.partial(
        pl.kernel,
        mesh=mesh,
        out_type=(
            jax.ShapeDtypeStruct((B, D), jnp.float32),
            jax.ShapeDtypeStruct((B, D), jnp.float32),
            jax.ShapeDtypeStruct((B, D), jnp.float32),
        ),
        scratch_types=[
            pltpu.VMEM((_HALF,), jnp.int32),
            pltpu.VMEM((_NFIRE, _CH), jnp.int32),
            pltpu.VMEM((_HALF, D), jnp.float32),
            pltpu.SemaphoreType.DMA,
        ],
    )
    def sc_gather(i1, i2, i5, dst, t1, t2, t5, o1, o2, o5,
                  idx_v, dst_v, rows_v, sem):
        wid = lax.axis_index("s") * _NC + lax.axis_index("c")
        base0 = wid * _BPW

        for ih, th, oh in ((i1, t1, o1), (i2, t2, o2), (i5, t5, o5)):
            def round_body(r, carry, ih=ih, th=th, oh=oh):
                base = base0 + r * _HALF
                pltpu.sync_copy(ih.at[pl.ds(base, _HALF)], idx_v)
                pltpu.sync_copy(dst.at[wid, r], dst_v)
                for j in range(_NFIRE):
                    pltpu.async_copy(
                        th.at[idx_v.at[pl.ds(j * _CH, _CH)]],
                        rows_v.at[pl.ds(j * _CH, _CH)],
                        sem,
                    )
                for j in range(_NFIRE):
                    pltpu.make_async_copy(
                        th.at[idx_v.at[pl.ds(j * _CH, _CH)]],
                        rows_v.at[pl.ds(j * _CH, _CH)],
                        sem,
                    ).wait()
                for j in range(_NFIRE):
                    pltpu.async_copy(
                        rows_v.at[pl.ds(j * _CH, _CH)],
                        oh.at[dst_v.at[j]],
                        sem,
                    )
                for j in range(_NFIRE):
                    pltpu.make_async_copy(
                        rows_v.at[pl.ds(j * _CH, _CH)],
                        oh.at[dst_v.at[j]],
                        sem,
                    ).wait()
                return carry

            lax.fori_loop(0, _BPW // _HALF, round_body, 0)

    return sc_gather


_GRID = L


def _tc_body(x3, l3, x4, l4, x6, l6, wk, dy, g1, g2, g5,
             arow, brow, crow, srow, w3cat, w4cat, w6cat, b3, b4, b6, cdt,
             out_ref):
    f32 = jnp.float32
    j = pl.program_id(0)

    e = (lax.broadcasted_iota(jnp.int32, (L, 1), 0) == j).astype(f32)
    col = lambda ref: jnp.dot(ref[...], e, preferred_element_type=f32,
                              precision=lax.Precision.HIGHEST)

    z3 = col(x3) - col(l3)
    z4 = col(x4) - col(l4)
    z6 = col(x6) - col(l6)
    a = z3 * arow[...] + z4 * brow[...] + z6 * crow[...] + srow[...]
    f = jnp.sin(a)
    e3 = jnp.dot(f, w3cat[...], preferred_element_type=f32) + b3[...]
    e4 = jnp.dot(f, w4cat[...], preferred_element_type=f32) + b4[...]
    e6 = jnp.dot(f, w6cat[...], preferred_element_type=f32) + b6[...]

    wkc = col(wk).astype(jnp.int32)
    dyc = col(dy).astype(jnp.int32)
    lanes = lax.broadcasted_iota(jnp.int32, (N, 128), 1)
    oh = (lanes == wkc).astype(f32) + (lanes == dyc + 64).astype(f32)
    e6 += jnp.dot(oh, cdt[...], preferred_element_type=f32)

    out_ref[0, 0, :, :] = g1[...]
    out_ref[0, 1, :, :] = g2[...]
    out_ref[0, 2, :, :] = e3
    out_ref[0, 3, :, :] = e4
    out_ref[0, 4, :, :] = g5[...]
    out_ref[0, 5, :, :] = e6


def kernel(f1_lookup, f2_lookup, f3_content, f3_lookup, f4_content, f4_lookup,
           f5_lookup, f6_time, f6_lookup, f6_week, f6_day,
           emb1, emb2, W3, b3, W4, b4, emb5, W6, b6, week_tab, day_tab):
    i1 = f1_lookup.reshape(B).astype(jnp.int32)
    i2 = f2_lookup.reshape(B).astype(jnp.int32)
    i5 = f5_lookup.reshape(B).astype(jnp.int32)
    bs = jnp.arange(B, dtype=jnp.int32)
    dst = ((bs % L) * N + bs // L).reshape(_NW, _BPW // _HALF, _NFIRE, _CH)

    g1, g2, g5 = _make_sc_gather()(i1, i2, i5, dst, emb1, emb2, emb5)

    wc = _fourier_w(N_BANDS, 3)
    wt = _fourier_w(N_BANDS, 0)
    hp = math.pi / 2.0
    arow = jnp.zeros((1, 128), jnp.float32).at[0, 0:12].set(wc).at[0, 33:45].set(wc)
    brow = jnp.zeros((1, 128), jnp.float32).at[0, 12:24].set(wc).at[0, 45:57].set(wc)
    crow = jnp.zeros((1, 128), jnp.float32).at[0, 24:33].set(wt).at[0, 57:66].set(wt)
    srow = jnp.zeros((1, 128), jnp.float32).at[0, 33:66].set(hp)
    zw = jnp.zeros((128, D), jnp.float32)
    w3cat = zw.at[0:12].set(W3[:12]).at[33:45].set(W3[12:])
    w4cat = zw.at[12:24].set(W4[:12]).at[45:57].set(W4[12:])
    w6cat = zw.at[24:33].set(W6[:9]).at[57:66].set(W6[9:])
    cdt = zw.at[:57].set(week_tab).at[64:75].set(day_tab)

    nat_spec = pl.BlockSpec((N, L), lambda i: (0, 0))
    row_spec = pl.BlockSpec((N, D), lambda i: (i, 0))
    w_spec = lambda r: pl.BlockSpec((r, 128), lambda i: (0, 0))
    f32c = lambda a: a.astype(jnp.float32)

    out = pl.pallas_call(
        _tc_body,
        grid=(_GRID,),
        in_specs=[nat_spec] * 8 + [row_spec] * 3
        + [w_spec(1)] * 4 + [w_spec(128)] * 3 + [w_spec(1)] * 3 + [w_spec(128)],
        out_specs=pl.BlockSpec((1, 6, N, D), lambda i: (i, 0, 0, 0)),
        out_shape=jax.ShapeDtypeStruct((L, 6, N, D), jnp.float32),
    )(
        f32c(f3_content), f32c(f3_lookup),
        f32c(f4_content), f32c(f4_lookup),
        f32c(f6_time), f32c(f6_lookup),
        f32c(f6_week), f32c(f6_day),
        g1, g2, g5,
        arow, brow, crow, srow, w3cat, w4cat, w6cat,
        b3.reshape(1, D), b4.reshape(1, D), b6.reshape(1, D), cdt,
    )
    return jnp.transpose(out, (2, 0, 1, 3))


# speedup vs baseline: 1.0241x; 1.0241x over previous
import functools
import math

import jax
import jax.numpy as jnp
from jax import lax
from jax.experimental import pallas as pl
from jax.experimental.pallas import tpu as pltpu
from jax.experimental.pallas import tpu_sc as plsc

N, L = 1024, 50
B = N * L
D = 128
N_BANDS = 8

_NC, _NS = 2, 16
_NW = _NC * _NS
_BPW = B // _NW
_HALF = 800
_CH = 80
_NFIRE = _HALF // _CH
_NCH = _BPW // _CH


def _fourier_w(n_bands, offset):
    steps = n_bands + offset + 1
    w = 2.0 ** jnp.linspace(-float(n_bands), float(offset), steps)
    return (w * math.pi).astype(jnp.float32)


def _make_sc_gather():
    mesh = plsc.VectorSubcoreMesh(core_axis_name="c", subcore_axis_name="s")

    @functools.partial(
        pl.kernel,
        mesh=mesh,
        out_type=(
            jax.ShapeDtypeStruct((B, D), jnp.float32),
            jax.ShapeDtypeStruct((B, D), jnp.float32),
            jax.ShapeDtypeStruct((B, D), jnp.float32),
        ),
        scratch_types=[
            pltpu.VMEM((_HALF,), jnp.int32),
            pltpu.VMEM((_HALF, D), jnp.float32),
            pltpu.SemaphoreType.DMA,
        ],
    )
    def sc_gather(i1, i2, i5, t1, t2, t5, o1, o2, o5, idx_v, rows_v, sem):
        wid = lax.axis_index("s") * _NC + lax.axis_index("c")
        base0 = wid * _BPW

        for ih, th, oh in ((i1, t1, o1), (i2, t2, o2), (i5, t5, o5)):
            def round_body(r, carry, ih=ih, th=th, oh=oh):
                base = base0 + r * _HALF
                pltpu.sync_copy(ih.at[pl.ds(base, _HALF)], idx_v)
                for j in range(_NFIRE):
                    pltpu.async_copy(
                        th.at[idx_v.at[pl.ds(j * _CH, _CH)]],
                        rows_v.at[pl.ds(j * _CH, _CH)],
                        sem,
                    )
                for j in range(_NFIRE):
                    pltpu.make_async_copy(
                        th.at[idx_v.at[pl.ds(j * _CH, _CH)]],
                        rows_v.at[pl.ds(j * _CH, _CH)],
                        sem,
                    ).wait()
                pltpu.sync_copy(rows_v, oh.at[pl.ds(base, _HALF)])
                return carry

            lax.fori_loop(0, _BPW // _HALF, round_body, 0)

    return sc_gather


_GRID = L


def _tc_body(x3, l3, x4, l4, x6, l6, wk, dy, g1, g2, g5,
             arow, brow, crow, srow, w3cat, w4cat, w6cat, b3, b4, b6, cdt,
             out_ref):
    f32 = jnp.float32
    j = pl.program_id(0)

    e = (lax.broadcasted_iota(jnp.int32, (L, 1), 0) == j).astype(f32)
    col = lambda ref: jnp.dot(ref[...], e, preferred_element_type=f32,
                              precision=lax.Precision.HIGHEST)

    z3 = col(x3) - col(l3)
    z4 = col(x4) - col(l4)
    z6 = col(x6) - col(l6)
    a = z3 * arow[...] + z4 * brow[...] + z6 * crow[...] + srow[...]
    f = jnp.sin(a)
    e3 = jnp.dot(f, w3cat[...], preferred_element_type=f32) + b3[...]
    e4 = jnp.dot(f, w4cat[...], preferred_element_type=f32) + b4[...]
    e6 = jnp.dot(f, w6cat[...], preferred_element_type=f32) + b6[...]

    wkc = col(wk).astype(jnp.int32)
    dyc = col(dy).astype(jnp.int32)
    lanes = lax.broadcasted_iota(jnp.int32, (N, 128), 1)
    oh = (lanes == wkc).astype(f32) + (lanes == dyc + 64).astype(f32)
    e6 += jnp.dot(oh, cdt[...], preferred_element_type=f32)

    out_ref[0, 0, :, :] = g1[...]
    out_ref[0, 1, :, :] = g2[...]
    out_ref[0, 2, :, :] = e3
    out_ref[0, 3, :, :] = e4
    out_ref[0, 4, :, :] = g5[...]
    out_ref[0, 5, :, :] = e6


def kernel(f1_lookup, f2_lookup, f3_content, f3_lookup, f4_content, f4_lookup,
           f5_lookup, f6_time, f6_lookup, f6_week, f6_day,
           emb1, emb2, W3, b3, W4, b4, emb5, W6, b6, week_tab, day_tab):
    i1 = f1_lookup.T.reshape(B).astype(jnp.int32)
    i2 = f2_lookup.T.reshape(B).astype(jnp.int32)
    i5 = f5_lookup.T.reshape(B).astype(jnp.int32)

    g1, g2, g5 = _make_sc_gather()(i1, i2, i5, emb1, emb2, emb5)

    wc = _fourier_w(N_BANDS, 3)
    wt = _fourier_w(N_BANDS, 0)
    hp = math.pi / 2.0
    arow = jnp.zeros((1, 128), jnp.float32).at[0, 0:12].set(wc).at[0, 33:45].set(wc)
    brow = jnp.zeros((1, 128), jnp.float32).at[0, 12:24].set(wc).at[0, 45:57].set(wc)
    crow = jnp.zeros((1, 128), jnp.float32).at[0, 24:33].set(wt).at[0, 57:66].set(wt)
    srow = jnp.zeros((1, 128), jnp.float32).at[0, 33:66].set(hp)
    zw = jnp.zeros((128, D), jnp.float32)
    w3cat = zw.at[0:12].set(W3[:12]).at[33:45].set(W3[12:])
    w4cat = zw.at[12:24].set(W4[:12]).at[45:57].set(W4[12:])
    w6cat = zw.at[24:33].set(W6[:9]).at[57:66].set(W6[9:])
    cdt = zw.at[:57].set(week_tab).at[64:75].set(day_tab)

    nat_spec = pl.BlockSpec((N, L), lambda i: (0, 0))
    row_spec = pl.BlockSpec((N, D), lambda i: (i, 0))
    w_spec = lambda r: pl.BlockSpec((r, 128), lambda i: (0, 0))
    f32c = lambda a: a.astype(jnp.float32)

    out = pl.pallas_call(
        _tc_body,
        grid=(_GRID,),
        in_specs=[nat_spec] * 8 + [row_spec] * 3
        + [w_spec(1)] * 4 + [w_spec(128)] * 3 + [w_spec(1)] * 3 + [w_spec(128)],
        out_specs=pl.BlockSpec((1, 6, N, D), lambda i: (i, 0, 0, 0)),
        out_shape=jax.ShapeDtypeStruct((L, 6, N, D), jnp.float32),
    )(
        f32c(f3_content), f32c(f3_lookup),
        f32c(f4_content), f32c(f4_lookup),
        f32c(f6_time), f32c(f6_lookup),
        f32c(f6_week), f32c(f6_day),
        g1, g2, g5,
        arow, brow, crow, srow, w3cat, w4cat, w6cat,
        b3.reshape(1, D), b4.reshape(1, D), b6.reshape(1, D), cdt,
    )
    return jnp.transpose(out, (2, 0, 1, 3))


# speedup vs baseline: 1.8421x; 1.7986x over previous
import functools
import math

import jax
import jax.numpy as jnp
from jax import lax
from jax.experimental import pallas as pl
from jax.experimental.pallas import tpu as pltpu
from jax.experimental.pallas import tpu_sc as plsc

N, L = 1024, 50
B = N * L
D = 128
N_BANDS = 8

_NC, _NS = 2, 16
_NW = _NC * _NS
_BPW = B // _NW
_HALF = 800
_CH = 80
_NFIRE = _HALF // _CH
_NCH = _BPW // _CH


def _fourier_w(n_bands, offset):
    steps = n_bands + offset + 1
    w = 2.0 ** jnp.linspace(-float(n_bands), float(offset), steps)
    return (w * math.pi).astype(jnp.float32)


def _make_sc_gather():
    mesh = plsc.VectorSubcoreMesh(core_axis_name="c", subcore_axis_name="s")

    @functools.partial(
        pl.kernel,
        mesh=mesh,
        out_type=(
            jax.ShapeDtypeStruct((B, D), jnp.float32),
            jax.ShapeDtypeStruct((B, D), jnp.float32),
            jax.ShapeDtypeStruct((B, D), jnp.float32),
        ),
        scratch_types=[
            pltpu.VMEM((_HALF,), jnp.int32),
            pltpu.VMEM((_HALF, D), jnp.float32),
            pltpu.SemaphoreType.DMA,
        ],
    )
    def sc_gather(i1, i2, i5, t1, t2, t5, o1, o2, o5, idx_v, rows_v, sem):
        wid = lax.axis_index("s") * _NC + lax.axis_index("c")
        base0 = wid * _BPW

        for ih, th, oh in ((i1, t1, o1), (i2, t2, o2), (i5, t5, o5)):
            def round_body(r, carry, ih=ih, th=th, oh=oh):
                base = base0 + r * _HALF
                pltpu.sync_copy(ih.at[pl.ds(base, _HALF)], idx_v)
                for j in range(_NFIRE):
                    pltpu.async_copy(
                        th.at[idx_v.at[pl.ds(j * _CH, _CH)]],
                        rows_v.at[pl.ds(j * _CH, _CH)],
                        sem,
                    )
                for j in range(_NFIRE):
                    pltpu.make_async_copy(
                        th.at[idx_v.at[pl.ds(j * _CH, _CH)]],
                        rows_v.at[pl.ds(j * _CH, _CH)],
                        sem,
                    ).wait()
                pltpu.sync_copy(rows_v, oh.at[pl.ds(base, _HALF)])
                return carry

            lax.fori_loop(0, _BPW // _HALF, round_body, 0)

    return sc_gather


_GRID = L


def _tc_body(x3, l3, x4, l4, x6, l6, wk, dy, g1, g2, g5,
             acol, bcol, ccol, scol, w3cat, w4cat, w6cat, b3, b4, b6, cdt,
             out_ref):
    f32 = jnp.float32
    dim0 = (((0,), (0,)), ((), ()))

    z3 = x3[0] - l3[0]
    z4 = x4[0] - l4[0]
    z6 = x6[0] - l6[0]
    a = acol[...] * z3 + bcol[...] * z4 + ccol[...] * z6 + scol[...]
    f = jnp.sin(a)
    e3 = lax.dot_general(f, w3cat[...], dim0, preferred_element_type=f32) + b3[...]
    e4 = lax.dot_general(f, w4cat[...], dim0, preferred_element_type=f32) + b4[...]
    e6 = lax.dot_general(f, w6cat[...], dim0, preferred_element_type=f32) + b6[...]

    rows = lax.broadcasted_iota(jnp.int32, (128, N), 0)
    oh = (rows == wk[0]).astype(f32) + (rows == dy[0] + 64).astype(f32)
    e6 += lax.dot_general(oh, cdt[...], dim0, preferred_element_type=f32)

    out_ref[0, 0, :, :] = g1[...]
    out_ref[0, 1, :, :] = g2[...]
    out_ref[0, 2, :, :] = e3
    out_ref[0, 3, :, :] = e4
    out_ref[0, 4, :, :] = g5[...]
    out_ref[0, 5, :, :] = e6


def kernel(f1_lookup, f2_lookup, f3_content, f3_lookup, f4_content, f4_lookup,
           f5_lookup, f6_time, f6_lookup, f6_week, f6_day,
           emb1, emb2, W3, b3, W4, b4, emb5, W6, b6, week_tab, day_tab):
    i1 = f1_lookup.T.reshape(B).astype(jnp.int32)
    i2 = f2_lookup.T.reshape(B).astype(jnp.int32)
    i5 = f5_lookup.T.reshape(B).astype(jnp.int32)

    g1, g2, g5 = _make_sc_gather()(i1, i2, i5, emb1, emb2, emb5)

    wc = _fourier_w(N_BANDS, 3)
    wt = _fourier_w(N_BANDS, 0)
    hp = math.pi / 2.0
    acol = jnp.zeros((128, 1), jnp.float32).at[0:12, 0].set(wc).at[33:45, 0].set(wc)
    bcol = jnp.zeros((128, 1), jnp.float32).at[12:24, 0].set(wc).at[45:57, 0].set(wc)
    ccol = jnp.zeros((128, 1), jnp.float32).at[24:33, 0].set(wt).at[57:66, 0].set(wt)
    scol = jnp.zeros((128, 1), jnp.float32).at[33:66, 0].set(hp)
    zw = jnp.zeros((128, D), jnp.float32)
    w3cat = zw.at[0:12].set(W3[:12]).at[33:45].set(W3[12:])
    w4cat = zw.at[12:24].set(W4[:12]).at[45:57].set(W4[12:])
    w6cat = zw.at[24:33].set(W6[:9]).at[57:66].set(W6[9:])
    cdt = zw.at[:57].set(week_tab).at[64:75].set(day_tab)

    col_spec = pl.BlockSpec((1, 1, N), lambda i: (i, 0, 0))
    row_spec = pl.BlockSpec((N, D), lambda i: (i, 0))
    w_spec = lambda r: pl.BlockSpec((r, 128), lambda i: (0, 0))
    cw_spec = pl.BlockSpec((128, 1), lambda i: (0, 0))
    lmaj = lambda x, dt: x.T.reshape(L, 1, N).astype(dt)

    out = pl.pallas_call(
        _tc_body,
        grid=(_GRID,),
        in_specs=[col_spec] * 8 + [row_spec] * 3
        + [cw_spec] * 4 + [w_spec(128)] * 3 + [w_spec(1)] * 3 + [w_spec(128)],
        out_specs=pl.BlockSpec((1, 6, N, D), lambda i: (i, 0, 0, 0)),
        out_shape=jax.ShapeDtypeStruct((L, 6, N, D), jnp.float32),
    )(
        lmaj(f3_content, jnp.float32), lmaj(f3_lookup, jnp.float32),
        lmaj(f4_content, jnp.float32), lmaj(f4_lookup, jnp.float32),
        lmaj(f6_time, jnp.float32), lmaj(f6_lookup, jnp.float32),
        lmaj(f6_week, jnp.int32), lmaj(f6_day, jnp.int32),
        g1, g2, g5,
        acol, bcol, ccol, scol, w3cat, w4cat, w6cat,
        b3.reshape(1, D), b4.reshape(1, D), b6.reshape(1, D), cdt,
    )
    return jnp.transpose(out, (2, 0, 1, 3))


# speedup vs baseline: 2.1977x; 1.1931x over previous
import functools
import math

import jax
import jax.numpy as jnp
from jax import lax
from jax.experimental import pallas as pl
from jax.experimental.pallas import tpu as pltpu
from jax.experimental.pallas import tpu_sc as plsc

N, L = 1024, 50
B = N * L
D = 128
N_BANDS = 8

_NC, _NS = 2, 16
_NW = _NC * _NS
_BPW = B // _NW
_HALF = 800
_CH = 80
_NFIRE = _HALF // _CH
_NCH = _BPW // _CH


def _fourier_w(n_bands, offset):
    steps = n_bands + offset + 1
    w = 2.0 ** jnp.linspace(-float(n_bands), float(offset), steps)
    return w.astype(jnp.float32)


_SC1, _SC3, _SC5, _SC7, _SC9 = (
    3.1415926, -5.16770808, 2.55005102, -0.59816166, 0.07744729)


def _sin_halfturns(s):
    k = jnp.round(s)
    r = s - k
    r2 = r * r
    p = _SC9
    p = p * r2 + _SC7
    p = p * r2 + _SC5
    p = p * r2 + _SC3
    p = p * r2 + _SC1
    v = p * r
    odd = (k.astype(jnp.int32) & 1) == 1
    return jnp.where(odd, -v, v)


def _make_sc_gather():
    mesh = plsc.VectorSubcoreMesh(core_axis_name="c", subcore_axis_name="s")

    @functools.partial(
        pl.kernel,
        mesh=mesh,
        out_type=(
            jax.ShapeDtypeStruct((B, D), jnp.float32),
            jax.ShapeDtypeStruct((B, D), jnp.float32),
            jax.ShapeDtypeStruct((B, D), jnp.float32),
        ),
        scratch_types=[
            pltpu.VMEM((_HALF,), jnp.int32),
            pltpu.VMEM((_HALF, D), jnp.float32),
            pltpu.SemaphoreType.DMA,
        ],
    )
    def sc_gather(i1, i2, i5, t1, t2, t5, o1, o2, o5, idx_v, rows_v, sem):
        wid = lax.axis_index("s") * _NC + lax.axis_index("c")
        base0 = wid * _BPW

        for ih, th, oh in ((i1, t1, o1), (i2, t2, o2), (i5, t5, o5)):
            def round_body(r, carry, ih=ih, th=th, oh=oh):
                base = base0 + r * _HALF
                pltpu.sync_copy(ih.at[pl.ds(base, _HALF)], idx_v)
                for j in range(_NFIRE):
                    pltpu.async_copy(
                        th.at[idx_v.at[pl.ds(j * _CH, _CH)]],
                        rows_v.at[pl.ds(j * _CH, _CH)],
                        sem,
                    )
                for j in range(_NFIRE):
                    pltpu.make_async_copy(
                        th.at[idx_v.at[pl.ds(j * _CH, _CH)]],
                        rows_v.at[pl.ds(j * _CH, _CH)],
                        sem,
                    ).wait()
                pltpu.sync_copy(rows_v, oh.at[pl.ds(base, _HALF)])
                return carry

            lax.fori_loop(0, _BPW // _HALF, round_body, 0)

    return sc_gather


_GRID = L


def _tc_body(x3, l3, x4, l4, x6, l6, wk, dy, g1, g2, g5,
             acol, bcol, ccol, scol, w3cat, w4cat, w6cat, b3, b4, b6, cdt,
             out_ref):
    f32 = jnp.float32
    dim0 = (((0,), (0,)), ((), ()))

    z3 = x3[0] - l3[0]
    z4 = x4[0] - l4[0]
    z6 = x6[0] - l6[0]
    a = acol[...] * z3 + bcol[...] * z4 + ccol[...] * z6 + scol[...]
    f = _sin_halfturns(a)
    e3 = lax.dot_general(f, w3cat[...], dim0, preferred_element_type=f32) + b3[...]
    e4 = lax.dot_general(f, w4cat[...], dim0, preferred_element_type=f32) + b4[...]
    e6 = lax.dot_general(f, w6cat[...], dim0, preferred_element_type=f32) + b6[...]

    rows = lax.broadcasted_iota(jnp.int32, (128, N), 0)
    oh = (rows == wk[0]).astype(f32) + (rows == dy[0] + 64).astype(f32)
    e6 += lax.dot_general(oh, cdt[...], dim0, preferred_element_type=f32)

    out_ref[0, 0, :, :] = g1[...]
    out_ref[0, 1, :, :] = g2[...]
    out_ref[0, 2, :, :] = e3
    out_ref[0, 3, :, :] = e4
    out_ref[0, 4, :, :] = g5[...]
    out_ref[0, 5, :, :] = e6


def kernel(f1_lookup, f2_lookup, f3_content, f3_lookup, f4_content, f4_lookup,
           f5_lookup, f6_time, f6_lookup, f6_week, f6_day,
           emb1, emb2, W3, b3, W4, b4, emb5, W6, b6, week_tab, day_tab):
    i1 = f1_lookup.T.reshape(B).astype(jnp.int32)
    i2 = f2_lookup.T.reshape(B).astype(jnp.int32)
    i5 = f5_lookup.T.reshape(B).astype(jnp.int32)

    g1, g2, g5 = _make_sc_gather()(i1, i2, i5, emb1, emb2, emb5)

    wc = _fourier_w(N_BANDS, 3)
    wt = _fourier_w(N_BANDS, 0)
    acol = jnp.zeros((128, 1), jnp.float32).at[0:12, 0].set(wc).at[33:45, 0].set(wc)
    bcol = jnp.zeros((128, 1), jnp.float32).at[12:24, 0].set(wc).at[45:57, 0].set(wc)
    ccol = jnp.zeros((128, 1), jnp.float32).at[24:33, 0].set(wt).at[57:66, 0].set(wt)
    scol = jnp.zeros((128, 1), jnp.float32).at[33:66, 0].set(0.5)
    zw = jnp.zeros((128, D), jnp.float32)
    w3cat = zw.at[0:12].set(W3[:12]).at[33:45].set(W3[12:])
    w4cat = zw.at[12:24].set(W4[:12]).at[45:57].set(W4[12:])
    w6cat = zw.at[24:33].set(W6[:9]).at[57:66].set(W6[9:])
    cdt = zw.at[:57].set(week_tab).at[64:75].set(day_tab)

    col_spec = pl.BlockSpec((1, 1, N), lambda i: (i, 0, 0))
    row_spec = pl.BlockSpec((N, D), lambda i: (i, 0))
    w_spec = lambda r: pl.BlockSpec((r, 128), lambda i: (0, 0))
    cw_spec = pl.BlockSpec((128, 1), lambda i: (0, 0))
    lmaj = lambda x, dt: x.T.reshape(L, 1, N).astype(dt)

    out = pl.pallas_call(
        _tc_body,
        grid=(_GRID,),
        in_specs=[col_spec] * 8 + [row_spec] * 3
        + [cw_spec] * 4 + [w_spec(128)] * 3 + [w_spec(1)] * 3 + [w_spec(128)],
        out_specs=pl.BlockSpec((1, 6, N, D), lambda i: (i, 0, 0, 0)),
        out_shape=jax.ShapeDtypeStruct((L, 6, N, D), jnp.float32),
    )(
        lmaj(f3_content, jnp.float32), lmaj(f3_lookup, jnp.float32),
        lmaj(f4_content, jnp.float32), lmaj(f4_lookup, jnp.float32),
        lmaj(f6_time, jnp.float32), lmaj(f6_lookup, jnp.float32),
        lmaj(f6_week, jnp.int32), lmaj(f6_day, jnp.int32),
        g1, g2, g5,
        acol, bcol, ccol, scol, w3cat, w4cat, w6cat,
        b3.reshape(1, D), b4.reshape(1, D), b6.reshape(1, D), cdt,
    )
    return jnp.transpose(out, (2, 0, 1, 3))
